# Initial kernel scaffold; baseline (speedup 1.0000x reference)
import functools

import jax
import jax.numpy as jnp
from jax import lax
from jax.experimental import pallas as pl
from jax.experimental.pallas import tpu as pltpu
from jax.experimental.pallas import tpu_sc as plsc


def _argmin_body(z_ref, cb_ref, tok_ref):
    z = z_ref[...]
    cb = cb_ref[...]
    d3 = 2.0 * lax.dot_general(
        z, cb, (((1,), (1,)), ((), ())),
        preferred_element_type=jnp.float32,
    )
    d1 = jnp.sum(z * z, axis=1, keepdims=True)
    d2 = jnp.sum(cb * cb, axis=1)
    dist = d1 + d2 - d3
    tok = jnp.argmin(dist, axis=1).astype(jnp.int32)
    tok_ref[...] = tok.reshape(tok_ref.shape)


def _compute_tokens(z_flat, codebook, block_n):
    n, d = z_flat.shape
    k = codebook.shape[0]
    nb = n // block_n
    tok = pl.pallas_call(
        _argmin_body,
        grid=(nb,),
        in_specs=[
            pl.BlockSpec((block_n, d), lambda i: (i, 0)),
            pl.BlockSpec((k, d), lambda i: (0, 0)),
        ],
        out_specs=pl.BlockSpec((1, 1, block_n), lambda i: (i, 0, 0)),
        out_shape=jax.ShapeDtypeStruct((nb, 1, block_n), jnp.int32),
    )(z_flat, codebook)
    return tok.reshape(n)


_SC_CHUNK = 128


def _make_sc_gather(k, d, n):
    info = plsc.get_sparse_core_info()
    nc, ns = info.num_cores, info.num_subcores
    nw = nc * ns
    b_per_w = n // nw
    chunks = b_per_w // _SC_CHUNK
    mesh = plsc.VectorSubcoreMesh(core_axis_name="c", subcore_axis_name="s")

    @functools.partial(
        pl.kernel,
        mesh=mesh,
        out_type=jax.ShapeDtypeStruct((n, d), jnp.float32),
        scratch_types=[
            pltpu.VMEM((chunks, _SC_CHUNK), jnp.int32),
            pltpu.VMEM((b_per_w, d), jnp.float32),
            pltpu.SemaphoreType.DMA,
        ],
    )
    def sc_gather(table_hbm, idx_hbm, out_hbm, idx_v, rows_v, sem):
        wid = lax.axis_index("s") * nc + lax.axis_index("c")
        base = wid * b_per_w
        for j in range(chunks):
            pltpu.sync_copy(
                idx_hbm.at[pl.ds(base + j * _SC_CHUNK, _SC_CHUNK)],
                idx_v.at[j],
            )
        copies = []
        for j in range(chunks):
            copies.append(
                pltpu.async_copy(
                    table_hbm.at[idx_v.at[j]],
                    rows_v.at[pl.ds(j * _SC_CHUNK, _SC_CHUNK)],
                    sem,
                )
            )
        for c in copies:
            c.wait()
        pltpu.sync_copy(rows_v, out_hbm.at[pl.ds(base, b_per_w)])

    return sc_gather


def kernel(z, codebook):
    k, d = codebook.shape
    z_flat = z.reshape(-1, d)
    n = z_flat.shape[0]
    tokens = _compute_tokens(z_flat, codebook, block_n=256)
    z_q = _make_sc_gather(k, d, n)(codebook, tokens)
    return (
        z_q.reshape(z.shape),
        tokens.reshape(z.shape[:-1]),
        codebook,
    )


# speedup vs baseline: 1.1522x; 1.1522x over previous
import functools

import jax
import jax.numpy as jnp
from jax import lax
from jax.experimental import pallas as pl
from jax.experimental.pallas import tpu as pltpu
from jax.experimental.pallas import tpu_sc as plsc

_BN = 256
_KB = 1024


def _argmin_body(cb_ref, zb_ref, d1_ref, d2_ref, tok_ref):
    k_total = cb_ref.shape[0]
    k_half = k_total // 2
    zb = zb_ref[...]
    d1 = d1_ref[...][0]
    half_m = []
    half_i = []
    for h in range(2):
        run_m = jnp.full((1, _BN), jnp.inf, jnp.float32)
        run_i = jnp.zeros((1, _BN), jnp.int32)
        for c in range(k_half // _KB):
            k0 = h * k_half + c * _KB
            cb = cb_ref[pl.ds(k0, _KB), :]
            d2 = d2_ref[0, 0, pl.ds(k0, _KB)]
            dT = lax.dot_general(cb, zb, (((1,), (1,)), ((), ())),
                                 preferred_element_type=jnp.float32)
            dist = (d1 + d2[:, None]) - 2.0 * dT
            m = jnp.min(dist, axis=0, keepdims=True)
            kio = lax.broadcasted_iota(jnp.int32, (_KB, _BN), 0) + k0
            idx = jnp.min(jnp.where(dist == m, kio, k_total),
                          axis=0, keepdims=True)
            upd = m < run_m
            run_i = jnp.where(upd, idx, run_i)
            run_m = jnp.where(upd, m, run_m)
        half_m.append(run_m)
        half_i.append(run_i)
    m0b = half_m[0].astype(jnp.bfloat16).astype(jnp.float32)
    upd = half_m[1] < m0b
    tok = jnp.where(upd, half_i[1], half_i[0])
    tok_ref[...] = tok.reshape(1, 1, _BN)


def _compute_tokens(z_flat, codebook):
    n, d = z_flat.shape
    k = codebook.shape[0]
    nb = n // _BN
    zb = z_flat.astype(jnp.bfloat16)
    d1 = jnp.sum(z_flat ** 2, axis=1).reshape(nb, 1, _BN)
    d2 = jnp.sum(codebook ** 2, axis=1).reshape(1, 1, k)
    tok = pl.pallas_call(
        _argmin_body,
        grid=(nb,),
        in_specs=[
            pl.BlockSpec((k, d), lambda i: (0, 0)),
            pl.BlockSpec((_BN, d), lambda i: (i, 0)),
            pl.BlockSpec((1, 1, _BN), lambda i: (i, 0, 0)),
            pl.BlockSpec((1, 1, k), lambda i: (0, 0, 0)),
        ],
        out_specs=pl.BlockSpec((1, 1, _BN), lambda i: (i, 0, 0)),
        out_shape=jax.ShapeDtypeStruct((nb, 1, _BN), jnp.int32),
    )(codebook, zb, d1, d2)
    return tok.reshape(n)


_SC_CHUNK = 128


def _make_sc_gather(k, d, n):
    info = plsc.get_sparse_core_info()
    nc, ns = info.num_cores, info.num_subcores
    nw = nc * ns
    b_per_w = n // nw
    chunks = b_per_w // _SC_CHUNK
    mesh = plsc.VectorSubcoreMesh(core_axis_name="c", subcore_axis_name="s")

    @functools.partial(
        pl.kernel,
        mesh=mesh,
        compiler_params=pltpu.CompilerParams(use_tc_tiling_on_sc=False),
        out_type=jax.ShapeDtypeStruct((n, d), jnp.float32),
        scratch_types=[
            pltpu.VMEM((chunks, _SC_CHUNK), jnp.int32),
            pltpu.VMEM((b_per_w, d), jnp.float32),
            pltpu.SemaphoreType.DMA,
        ],
    )
    def sc_gather(table_hbm, idx_hbm, out_hbm, idx_v, rows_v, sem):
        wid = lax.axis_index("s") * nc + lax.axis_index("c")
        base = wid * b_per_w
        for j in range(chunks):
            pltpu.sync_copy(
                idx_hbm.at[pl.ds(base + j * _SC_CHUNK, _SC_CHUNK)],
                idx_v.at[j],
            )
        copies = []
        for j in range(chunks):
            copies.append(
                pltpu.async_copy(
                    table_hbm.at[idx_v.at[j]],
                    rows_v.at[pl.ds(j * _SC_CHUNK, _SC_CHUNK)],
                    sem,
                )
            )
        for c in copies:
            c.wait()
        pltpu.sync_copy(rows_v, out_hbm.at[pl.ds(base, b_per_w)])

    return sc_gather


def kernel(z, codebook):
    k, d = codebook.shape
    z_flat = z.reshape(-1, d)
    n = z_flat.shape[0]
    tokens = _compute_tokens(z_flat, codebook)
    z_q = _make_sc_gather(k, d, n)(codebook, tokens)
    return (
        z_q.reshape(z.shape),
        tokens.reshape(z.shape[:-1]),
        codebook,
    )
